# Initial kernel scaffold; baseline (speedup 1.0000x reference)
import functools

import jax
import jax.numpy as jnp
from jax import lax
from jax.experimental import pallas as pl
from jax.experimental.pallas import tpu as pltpu
from jax.experimental.pallas import tpu_sc as plsc

VOCAB = 1000000
EMB = 32
BATCH = 16384
HIST = 200

NC = 2
NS = 16
NW = NC * NS
LANES = 16

HALF = HIST // 2
CH = 16
NBUF = 4
B_PER_W = BATCH // NW
N_CHUNKS = B_PER_W // CH
RED_UNROLL = 10


def _body(ids2_hbm, len_hbm, table_hbm, out_hbm,
          idx_v, gbuf, len_v, out_v, s0, s1, s2, s3):
    wid = lax.axis_index("s") * NC + lax.axis_index("c")
    base = wid * B_PER_W
    sems = [s0, s1, s2, s3]
    zero = jnp.zeros((LANES,), jnp.float32)

    def reduce_half(buf):
        hb = gbuf.at[buf]

        def step(k, accs):
            a0, a1, b0, b1 = accs
            for u in range(RED_UNROLL):
                i = k * RED_UNROLL + u
                lo = hb[i, pl.ds(0, LANES)]
                hi = hb[i, pl.ds(LANES, LANES)]
                if u % 2 == 0:
                    a0 = a0 + lo
                    a1 = a1 + hi
                else:
                    b0 = b0 + lo
                    b1 = b1 + hi
            return (a0, a1, b0, b1)

        a0, a1, b0, b1 = lax.fori_loop(
            0, HALF // RED_UNROLL, step, (zero, zero, zero, zero))
        return a0 + b0, a1 + b1

    def chunk_body(ci, carry):
        cb = base + ci * CH
        j2b = 2 * cb

        pltpu.sync_copy(ids2_hbm.at[pl.ds(j2b, 2 * CH)], idx_v)
        pltpu.sync_copy(len_hbm.at[pl.ds(cb, CH)], len_v)

        lv = jnp.maximum(len_v[...], 1).astype(jnp.float32)
        recip = 1.0 / lv

        copies = [None] * NBUF
        for b in range(NBUF):
            copies[b] = pltpu.async_copy(
                table_hbm.at[idx_v.at[b]], gbuf.at[b], sems[b])

        pend = None
        for j in range(2 * CH):
            buf = j % NBUF
            copies[buf].wait()
            p0, p1 = reduce_half(buf)
            nj = j + NBUF
            if nj < 2 * CH:
                copies[buf] = pltpu.async_copy(
                    table_hbm.at[idx_v.at[nj]], gbuf.at[buf], sems[buf])
            if j % 2 == 0:
                pend = (p0, p1)
            else:
                r = j // 2
                s0v = pend[0] + p0
                s1v = pend[1] + p1
                rr = jnp.take(recip, jnp.full((LANES,), r, jnp.int32),
                              mode="promise_in_bounds")
                out_v[r, pl.ds(0, LANES)] = s0v * rr
                out_v[r, pl.ds(LANES, LANES)] = s1v * rr

        pltpu.sync_copy(out_v, out_hbm.at[pl.ds(cb, CH)])
        return carry

    lax.fori_loop(0, N_CHUNKS, chunk_body, 0)


@jax.jit
def _pooled(ids2, lengths, table):
    mesh = plsc.VectorSubcoreMesh(core_axis_name="c", subcore_axis_name="s")
    f = pl.kernel(
        _body,
        out_type=jax.ShapeDtypeStruct((BATCH, EMB), jnp.float32),
        mesh=mesh,
        scratch_types=[
            pltpu.VMEM((2 * CH, HALF), jnp.int32),
            pltpu.VMEM((NBUF, HALF, EMB), jnp.float32),
            pltpu.VMEM((CH,), jnp.int32),
            pltpu.VMEM((CH, EMB), jnp.float32),
            pltpu.SemaphoreType.DMA,
            pltpu.SemaphoreType.DMA,
            pltpu.SemaphoreType.DMA,
            pltpu.SemaphoreType.DMA,
        ],
    )
    return f(ids2, lengths, table)


def kernel(ids, lengths, table):
    ids2 = ids.reshape(2 * BATCH, HALF)
    return _pooled(ids2, lengths, table)


# speedup vs baseline: 13.8889x; 13.8889x over previous
import functools

import jax
import jax.numpy as jnp
from jax import lax
from jax.experimental import pallas as pl
from jax.experimental.pallas import tpu as pltpu
from jax.experimental.pallas import tpu_sc as plsc

VOCAB = 1000000
EMB = 32
BATCH = 16384
HIST = 200

NC = 2
NS = 16
NW = NC * NS
LANES = 16

HALF = HIST // 2
CH = 16
NBUF = 4
B_PER_W = BATCH // NW
N_CHUNKS = B_PER_W // CH
RED_UNROLL = 10


def _bcast_lane(vec, r):
    idxv = jnp.full((LANES, 1), r, jnp.int32)
    dnums = lax.GatherDimensionNumbers(
        offset_dims=(), collapsed_slice_dims=(0,), start_index_map=(0,))
    return lax.gather(vec, idxv, dnums, slice_sizes=(1,),
                      mode=lax.GatherScatterMode.PROMISE_IN_BOUNDS)


def _body(ids2_hbm, len_hbm, table_hbm, out_hbm,
          idx_v, gbuf, len_v, out_v, s0, s1, s2, s3):
    wid = lax.axis_index("s") * NC + lax.axis_index("c")
    base = wid * B_PER_W
    sems = [s0, s1, s2, s3]
    zero = jnp.zeros((LANES,), jnp.float32)

    def reduce_half(buf):
        hb = gbuf.at[buf]

        def step(k, accs):
            a0, a1, b0, b1 = accs
            for u in range(RED_UNROLL):
                i = k * RED_UNROLL + u
                lo = hb[i, pl.ds(0, LANES)]
                hi = hb[i, pl.ds(LANES, LANES)]
                if u % 2 == 0:
                    a0 = a0 + lo
                    a1 = a1 + hi
                else:
                    b0 = b0 + lo
                    b1 = b1 + hi
            return (a0, a1, b0, b1)

        a0, a1, b0, b1 = lax.fori_loop(
            0, HALF // RED_UNROLL, step, (zero, zero, zero, zero))
        return a0 + b0, a1 + b1

    def chunk_body(ci, carry):
        cb = base + ci * CH
        j2b = 2 * cb

        pltpu.sync_copy(ids2_hbm.at[pl.ds(j2b, 2 * CH)], idx_v)
        pltpu.sync_copy(len_hbm.at[pl.ds(cb, CH)], len_v)

        lv = jnp.maximum(len_v[...], 1).astype(jnp.float32)
        recip = 1.0 / lv

        copies = [None] * NBUF
        for b in range(NBUF):
            copies[b] = pltpu.async_copy(
                table_hbm.at[idx_v.at[b]], gbuf.at[b], sems[b])

        pend = None
        for j in range(2 * CH):
            buf = j % NBUF
            copies[buf].wait()
            p0, p1 = reduce_half(buf)
            nj = j + NBUF
            if nj < 2 * CH:
                copies[buf] = pltpu.async_copy(
                    table_hbm.at[idx_v.at[nj]], gbuf.at[buf], sems[buf])
            if j % 2 == 0:
                pend = (p0, p1)
            else:
                r = j // 2
                s0v = pend[0] + p0
                s1v = pend[1] + p1
                rr = _bcast_lane(recip, r)
                out_v[r, pl.ds(0, LANES)] = s0v * rr
                out_v[r, pl.ds(LANES, LANES)] = s1v * rr

        pltpu.sync_copy(out_v, out_hbm.at[pl.ds(cb, CH)])
        return carry

    lax.fori_loop(0, N_CHUNKS, chunk_body, 0)


@jax.jit
def _pooled(ids2, lengths, table):
    mesh = plsc.VectorSubcoreMesh(core_axis_name="c", subcore_axis_name="s")
    f = pl.kernel(
        _body,
        out_type=jax.ShapeDtypeStruct((BATCH, EMB), jnp.float32),
        mesh=mesh,
        compiler_params=pltpu.CompilerParams(use_tc_tiling_on_sc=False),
        scratch_types=[
            pltpu.VMEM((2 * CH, HALF), jnp.int32),
            pltpu.VMEM((NBUF, HALF, EMB), jnp.float32),
            pltpu.VMEM((CH,), jnp.int32),
            pltpu.VMEM((CH, EMB), jnp.float32),
            pltpu.SemaphoreType.DMA,
            pltpu.SemaphoreType.DMA,
            pltpu.SemaphoreType.DMA,
            pltpu.SemaphoreType.DMA,
        ],
    )
    return f(ids2, lengths, table)


def kernel(ids, lengths, table):
    ids2 = ids.reshape(2 * BATCH, HALF)
    return _pooled(ids2, lengths, table)
